# baseline (device time: 87936 ns/iter reference)
import jax
import jax.numpy as jnp
from jax import lax
from jax.experimental import pallas as pl
from jax.experimental.pallas import tpu as pltpu

N_DEV = 8
B_, S_, D_, N_ = 8, 512, 512, 16
NC, L = 32, 16
TW = 16


def kernel(x, A, B, C):
    x4 = x.reshape(B_, NC, L, D_)
    B4 = B.reshape(B_, NC, L, N_)
    C4 = C.reshape(B_, NC, L, N_)

    def body(x_ref, A_ref, B_ref, C_ref, out_ref,
             H_ref, xt_ref, bt_ref, send_sems, recv_sems):
        my = lax.axis_index("i")
        left = (my - 1) % N_DEV
        right = (my + 1) % N_DEV

        barrier_sem = pltpu.get_barrier_semaphore()
        for nbr in (left, right):
            pl.semaphore_signal(
                barrier_sem, inc=1,
                device_id=(nbr,), device_id_type=pl.DeviceIdType.MESH,
            )
        pl.semaphore_wait(barrier_sem, 2)

        @pl.when(my < N_DEV - 1)
        def _():
            for src, dst, i in ((x_ref, xt_ref, 0), (B_ref, bt_ref, 1)):
                rdma = pltpu.make_async_remote_copy(
                    src_ref=src.at[:, NC - 1, :, :], dst_ref=dst,
                    send_sem=send_sems.at[i], recv_sem=recv_sems.at[i],
                    device_id=(right,), device_id_type=pl.DeviceIdType.MESH,
                )
                rdma.start()

        dA_T = jnp.exp(A_ref[:, :]).T

        H_ref[...] = jnp.zeros((B_, NC, N_, D_), jnp.float32)

        def warm_main(w, _):
            xw = x_ref[:, 0:NC - 1, pl.ds(w, 1), :][:, :, 0, :]
            Bw = B_ref[:, 0:NC - 1, pl.ds(w, 1), :][:, :, 0, :]
            H_ref[:, 1:NC] = (H_ref[:, 1:NC] * dA_T[None, None]
                              + xw[:, :, None, :] * Bw[:, :, :, None])
            return 0

        lax.fori_loop(0, TW, warm_main, 0)

        @pl.when(my > 0)
        def _():
            for src, dst, i in ((x_ref, xt_ref, 0), (B_ref, bt_ref, 1)):
                rdma = pltpu.make_async_remote_copy(
                    src_ref=src.at[:, NC - 1, :, :], dst_ref=dst,
                    send_sem=send_sems.at[i], recv_sem=recv_sems.at[i],
                    device_id=(left,), device_id_type=pl.DeviceIdType.MESH,
                )
                rdma.wait_recv()

            def warm0(w, _):
                xw = xt_ref[:, pl.ds(w, 1), :]
                Bw = bt_ref[:, pl.ds(w, 1), :][:, 0, :]
                H_ref[:, 0:1] = (H_ref[:, 0:1] * dA_T[None, None]
                                 + xw[:, :, None, :]
                                 * Bw[:, None, :, None])
                return 0

            lax.fori_loop(0, TW, warm0, 0)

        def step(t, _):
            xt = x_ref[:, :, pl.ds(t, 1), :][:, :, 0, :]
            Bt = B_ref[:, :, pl.ds(t, 1), :][:, :, 0, :]
            Ct = C_ref[:, :, pl.ds(t, 1), :][:, :, 0, :]
            H = (H_ref[...] * dA_T[None, None]
                 + xt[:, :, None, :] * Bt[:, :, :, None])
            H_ref[...] = H
            yt = jnp.sum(H * Ct[:, :, :, None], axis=2)
            out_ref[:, :, pl.ds(t, 1), :] = yt[:, :, None, :]
            return 0

        lax.fori_loop(0, L, step, 0)

        @pl.when(my < N_DEV - 1)
        def _():
            for src, dst, i in ((x_ref, xt_ref, 0), (B_ref, bt_ref, 1)):
                rdma = pltpu.make_async_remote_copy(
                    src_ref=src.at[:, NC - 1, :, :], dst_ref=dst,
                    send_sem=send_sems.at[i], recv_sem=recv_sems.at[i],
                    device_id=(right,), device_id_type=pl.DeviceIdType.MESH,
                )
                rdma.wait_send()

    out4 = pl.pallas_call(
        body,
        out_shape=jax.ShapeDtypeStruct((B_, NC, L, D_), jnp.float32),
        in_specs=[
            pl.BlockSpec(memory_space=pltpu.VMEM),
            pl.BlockSpec(memory_space=pltpu.VMEM),
            pl.BlockSpec(memory_space=pltpu.VMEM),
            pl.BlockSpec(memory_space=pltpu.VMEM),
        ],
        out_specs=pl.BlockSpec(memory_space=pltpu.VMEM),
        scratch_shapes=[
            pltpu.VMEM((B_, NC, N_, D_), jnp.float32),
            pltpu.VMEM((B_, TW, D_), jnp.float32),
            pltpu.VMEM((B_, TW, N_), jnp.float32),
            pltpu.SemaphoreType.DMA((2,)),
            pltpu.SemaphoreType.DMA((2,)),
        ],
        compiler_params=pltpu.CompilerParams(
            collective_id=0, vmem_limit_bytes=100 * 1024 * 1024,
        ),
    )(x4, A, B4, C4)
    return out4.reshape(B_, S_, D_)


# device time: 86494 ns/iter; 1.0167x vs baseline; 1.0167x over previous
import jax
import jax.numpy as jnp
from jax import lax
from jax.experimental import pallas as pl
from jax.experimental.pallas import tpu as pltpu

N_DEV = 8
B_, S_, D_, N_ = 8, 512, 512, 16
NC, L = 32, 16
TW = 16


def kernel(x, A, B, C):
    x4 = x.reshape(B_, NC, L, D_)
    B4 = B.reshape(B_, NC, L, N_)
    C4 = C.reshape(B_, NC, L, N_)

    def body(x_ref, A_ref, B_ref, C_ref, out_ref,
             H_ref, xt_ref, bt_ref, send_sems, recv_sems):
        my = lax.axis_index("i")
        left = (my - 1) % N_DEV
        right = (my + 1) % N_DEV

        barrier_sem = pltpu.get_barrier_semaphore()
        for nbr in (left, right):
            pl.semaphore_signal(
                barrier_sem, inc=1,
                device_id=(nbr,), device_id_type=pl.DeviceIdType.MESH,
            )
        pl.semaphore_wait(barrier_sem, 2)

        @pl.when(my < N_DEV - 1)
        def _():
            for src, dst, i in ((x_ref, xt_ref, 0), (B_ref, bt_ref, 1)):
                rdma = pltpu.make_async_remote_copy(
                    src_ref=src.at[:, NC - 1, :, :], dst_ref=dst,
                    send_sem=send_sems.at[i], recv_sem=recv_sems.at[i],
                    device_id=(right,), device_id_type=pl.DeviceIdType.MESH,
                )
                rdma.start()

        dA_T = jnp.exp(A_ref[:, :]).T

        H_ref[...] = jnp.zeros((B_, NC, N_, D_), jnp.float32)

        Hw = jnp.zeros((B_, NC - 1, N_, D_), jnp.float32)
        for w in range(TW):
            xw = x_ref[:, 0:NC - 1, w, :]
            Bw = B_ref[:, 0:NC - 1, w, :]
            Hw = (Hw * dA_T[None, None]
                  + xw[:, :, None, :] * Bw[:, :, :, None])
        H_ref[:, 1:NC] = Hw

        @pl.when(my > 0)
        def _():
            for src, dst, i in ((x_ref, xt_ref, 0), (B_ref, bt_ref, 1)):
                rdma = pltpu.make_async_remote_copy(
                    src_ref=src.at[:, NC - 1, :, :], dst_ref=dst,
                    send_sem=send_sems.at[i], recv_sem=recv_sems.at[i],
                    device_id=(left,), device_id_type=pl.DeviceIdType.MESH,
                )
                rdma.wait_recv()

            H0 = jnp.zeros((B_, 1, N_, D_), jnp.float32)
            for w in range(TW):
                xw = xt_ref[:, w:w + 1, :]
                Bw = bt_ref[:, w, :]
                H0 = (H0 * dA_T[None, None]
                      + xw[:, :, None, :] * Bw[:, None, :, None])
            H_ref[:, 0:1] = H0

        H = H_ref[...]
        for t in range(L):
            xt = x_ref[:, :, t, :]
            Bt = B_ref[:, :, t, :]
            Ct = C_ref[:, :, t, :]
            H = (H * dA_T[None, None]
                 + xt[:, :, None, :] * Bt[:, :, :, None])
            yt = jnp.sum(H * Ct[:, :, :, None], axis=2)
            out_ref[:, :, t, :] = yt

        @pl.when(my < N_DEV - 1)
        def _():
            for src, dst, i in ((x_ref, xt_ref, 0), (B_ref, bt_ref, 1)):
                rdma = pltpu.make_async_remote_copy(
                    src_ref=src.at[:, NC - 1, :, :], dst_ref=dst,
                    send_sem=send_sems.at[i], recv_sem=recv_sems.at[i],
                    device_id=(right,), device_id_type=pl.DeviceIdType.MESH,
                )
                rdma.wait_send()

    out4 = pl.pallas_call(
        body,
        out_shape=jax.ShapeDtypeStruct((B_, NC, L, D_), jnp.float32),
        in_specs=[
            pl.BlockSpec(memory_space=pltpu.VMEM),
            pl.BlockSpec(memory_space=pltpu.VMEM),
            pl.BlockSpec(memory_space=pltpu.VMEM),
            pl.BlockSpec(memory_space=pltpu.VMEM),
        ],
        out_specs=pl.BlockSpec(memory_space=pltpu.VMEM),
        scratch_shapes=[
            pltpu.VMEM((B_, NC, N_, D_), jnp.float32),
            pltpu.VMEM((B_, TW, D_), jnp.float32),
            pltpu.VMEM((B_, TW, N_), jnp.float32),
            pltpu.SemaphoreType.DMA((2,)),
            pltpu.SemaphoreType.DMA((2,)),
        ],
        compiler_params=pltpu.CompilerParams(
            collective_id=0, vmem_limit_bytes=100 * 1024 * 1024,
        ),
    )(x4, A, B4, C4)
    return out4.reshape(B_, S_, D_)
